# baseline (speedup 1.0000x reference)
import functools

import jax
import jax.numpy as jnp
from jax import lax
from jax.experimental import pallas as pl
from jax.experimental.pallas import tpu as pltpu
from jax.experimental.pallas import tpu_sc as plsc

_B = 16384
_D = 32
_NC = 2
_NS = 16
_NW = _NC * _NS
_BPW = _B // _NW
_L = 16
_CHUNK = 128
_NCHUNK = _BPW // _CHUNK
_GROUPS = _BPW // _L


def _body(uid_hbm, iid_hbm, ut_hbm, it_hbm, out_hbm,
          uid_v, iid_v, ustage, vstage, out_v, sem):
    wid = lax.axis_index("s") * _NC + lax.axis_index("c")
    base = pl.multiple_of(wid * _BPW, _BPW)

    for c in range(_NCHUNK):
        pltpu.sync_copy(uid_hbm.at[pl.ds(base + c * _CHUNK, _CHUNK)],
                        uid_v.at[c])
        pltpu.sync_copy(iid_hbm.at[pl.ds(base + c * _CHUNK, _CHUNK)],
                        iid_v.at[c])

    def fetch(d, carry):
        doff = pl.multiple_of(d * _BPW, _CHUNK)
        for c in range(_NCHUNK):
            dst = pl.ds(doff + c * _CHUNK, _CHUNK)
            pltpu.async_copy(ut_hbm.at[d].at[uid_v.at[c]],
                             ustage.at[dst], sem)
            pltpu.async_copy(it_hbm.at[d].at[iid_v.at[c]],
                             vstage.at[dst], sem)
        return carry

    lax.fori_loop(0, _D, fetch, 0)

    pltpu.make_async_copy(out_hbm, ustage, sem).wait()
    pltpu.make_async_copy(out_hbm, vstage, sem).wait()

    def group(g, carry):
        goff = pl.multiple_of(g * _L, _L)
        accs = [jnp.zeros((_L,), jnp.float32) for _ in range(4)]
        for d in range(_D):
            off = pl.ds(pl.multiple_of(d * _BPW, _L) + goff, _L)
            accs[d % 4] = accs[d % 4] + ustage[off] * vstage[off]
        out_v[pl.ds(goff, _L)] = (accs[0] + accs[1]) + (accs[2] + accs[3])
        return carry

    lax.fori_loop(0, _GROUPS, group, 0)

    pltpu.sync_copy(out_v, out_hbm.at[pl.ds(base, _BPW)])


def kernel(user_ids, item_ids, user_factors, item_factors):
    mesh = plsc.VectorSubcoreMesh(core_axis_name="c", subcore_axis_name="s")
    k = functools.partial(
        pl.kernel,
        mesh=mesh,
        out_type=jax.ShapeDtypeStruct((_B,), jnp.float32),
        compiler_params=pltpu.CompilerParams(
            needs_layout_passes=False, use_tc_tiling_on_sc=False),
        scratch_types=[
            pltpu.VMEM((_NCHUNK, _CHUNK), jnp.int32),
            pltpu.VMEM((_NCHUNK, _CHUNK), jnp.int32),
            pltpu.VMEM((_D * _BPW,), jnp.float32),
            pltpu.VMEM((_D * _BPW,), jnp.float32),
            pltpu.VMEM((_BPW,), jnp.float32),
            pltpu.SemaphoreType.DMA,
        ],
    )(_body)
    return k(user_ids.astype(jnp.int32), item_ids.astype(jnp.int32),
             user_factors.T, item_factors.T)


# speedup vs baseline: 14.9053x; 14.9053x over previous
import functools

import jax
import jax.numpy as jnp
from jax import lax
from jax.experimental import pallas as pl
from jax.experimental.pallas import tpu as pltpu
from jax.experimental.pallas import tpu_sc as plsc

_B = 16384
_D = 32
_NC = 2
_NS = 16
_NW = _NC * _NS
_BPW = _B // _NW
_L = 16
_CHUNK = 128
_NCHUNK = _BPW // _CHUNK
_GROUPS = _BPW // _L

_NV = 1000000
_TILE = 128
_TPB = _NV // _TILE
_NFULL = _TPB * _TILE
_NTAIL = _NV - _NFULL
_G = 36
_CPB = _TPB // _G
_LCH = _G * _TILE
_UNITS = 4 * _CPB
_UPW = -(-_UNITS // _NW)
_NTILES = 4 * _TPB
_WORDS = _NTILES * 8 * _TILE
_KSTRIDE = _TPB * 8 * _TILE


def _repack_body(ut, it, ulin, ilin, bufu, bufi, sem):
    wid = lax.axis_index("s") * _NC + lax.axis_index("c")

    def unit(i, carry):
        u = wid + i * _NW

        @pl.when(u < _UNITS)
        def _():
            k = u // _CPB
            c = u % _CPB
            loff = pl.multiple_of(c * _LCH, _LCH)
            doff = pl.multiple_of(k * 8, 8)
            tile0 = k * _TPB + c * _G
            copies = []
            for j in range(_G):
                src = pl.ds(loff + j * _TILE, _TILE)
                copies.append(pltpu.async_copy(
                    ut.at[pl.ds(doff, 8), src], bufu.at[j], sem))
                copies.append(pltpu.async_copy(
                    it.at[pl.ds(doff, 8), src], bufi.at[j], sem))
            for cp in copies:
                cp.wait()
            wu = pltpu.async_copy(bufu, ulin.at[pl.ds(tile0, _G)], sem)
            wi = pltpu.async_copy(bufi, ilin.at[pl.ds(tile0, _G)], sem)
            wu.wait()
            wi.wait()

        return carry

    lax.fori_loop(0, _UPW, unit, 0)


def _gather_body(uid_hbm, iid_hbm, ulin, ilin, tailu_hbm, taili_hbm, out_hbm,
                 uid_v, iid_v, preu_v, prei_v, offu_v, offi_v, idx_v,
                 tailu_v, taili_v, ustage, vstage, out_v, sem):
    wid = lax.axis_index("s") * _NC + lax.axis_index("c")
    base = pl.multiple_of(wid * _BPW, _BPW)

    pltpu.sync_copy(tailu_hbm, tailu_v)
    pltpu.sync_copy(taili_hbm, taili_v)
    for c in range(_NCHUNK):
        pltpu.sync_copy(uid_hbm.at[pl.ds(base + c * _CHUNK, _CHUNK)],
                        uid_v.at[c])
        pltpu.sync_copy(iid_hbm.at[pl.ds(base + c * _CHUNK, _CHUNK)],
                        iid_v.at[c])

    def precomp(v, carry):
        s = pl.ds(pl.multiple_of(v * _L, _L), _L)
        c = v // (_CHUNK // _L)
        cs = pl.ds(pl.multiple_of((v % (_CHUNK // _L)) * _L, _L), _L)
        for ids, pre_v, off_v in ((uid_v, preu_v, offu_v),
                                  (iid_v, prei_v, offi_v)):
            r = ids[c, cs]
            off_v[s] = r - _NFULL
            rc = jnp.minimum(r, _NFULL - 1)
            pre_v[s] = ((rc >> 7) << 10) + (rc & 127)
        return carry

    lax.fori_loop(0, _BPW // _L, precomp, 0)

    def fetch(d, carry):
        k = d // 8
        q = d % 8
        dconst = k * _KSTRIDE + q * _TILE
        doff = pl.multiple_of(d * _BPW, _CHUNK)
        for c in range(_NCHUNK):
            for v in range(_CHUNK // _L):
                s = pl.ds(pl.multiple_of(v * _L, _L), _L)
                fs = pl.ds(pl.multiple_of(c * _CHUNK + v * _L, _L), _L)
                idx_v[0, s] = preu_v[fs] + dconst
                idx_v[1, s] = prei_v[fs] + dconst
            dst = pl.ds(doff + c * _CHUNK, _CHUNK)
            cu = pltpu.async_copy(ulin.at[idx_v.at[0]], ustage.at[dst], sem)
            ci = pltpu.async_copy(ilin.at[idx_v.at[1]], vstage.at[dst], sem)
            cu.wait()
            ci.wait()
        return carry

    lax.fori_loop(0, _D, fetch, 0)

    iota = lax.iota(jnp.int32, _L)
    i512 = iota * _BPW
    i64 = iota * _NTAIL

    def tailfix(g, carry):
        goff = pl.multiple_of(g * _L, _L)
        offu = offu_v[pl.ds(goff, _L)]
        offi = offi_v[pl.ds(goff, _L)]
        anytail = jnp.maximum(jnp.max(offu), jnp.max(offi))

        @pl.when(anytail >= 0)
        def _():
            for j in range(_L):
                b = goff + j

                @pl.when(offu[j] >= 0)
                def _():
                    src0 = plsc.load_gather(tailu_v, [i64 + offu[j]])
                    src1 = plsc.load_gather(tailu_v, [i64 + (offu[j] + _L * _NTAIL)])
                    plsc.store_scatter(ustage, [i512 + b], src0)
                    plsc.store_scatter(ustage, [i512 + (b + _L * _BPW)], src1)

                @pl.when(offi[j] >= 0)
                def _():
                    src0 = plsc.load_gather(taili_v, [i64 + offi[j]])
                    src1 = plsc.load_gather(taili_v, [i64 + (offi[j] + _L * _NTAIL)])
                    plsc.store_scatter(vstage, [i512 + b], src0)
                    plsc.store_scatter(vstage, [i512 + (b + _L * _BPW)], src1)

        return carry

    lax.fori_loop(0, _GROUPS, tailfix, 0)

    def group(g, carry):
        goff = pl.multiple_of(g * _L, _L)
        accs = [jnp.zeros((_L,), jnp.float32) for _ in range(4)]
        for d in range(_D):
            off = pl.ds(pl.multiple_of(d * _BPW, _L) + goff, _L)
            accs[d % 4] = accs[d % 4] + ustage[off] * vstage[off]
        out_v[pl.ds(goff, _L)] = (accs[0] + accs[1]) + (accs[2] + accs[3])
        return carry

    lax.fori_loop(0, _GROUPS, group, 0)

    pltpu.sync_copy(out_v, out_hbm.at[pl.ds(base, _BPW)])


def kernel(user_ids, item_ids, user_factors, item_factors):
    mesh = plsc.VectorSubcoreMesh(core_axis_name="c", subcore_axis_name="s")

    repack = functools.partial(
        pl.kernel,
        mesh=mesh,
        out_type=(jax.ShapeDtypeStruct((_NTILES, 8, _TILE), jnp.float32),
                  jax.ShapeDtypeStruct((_NTILES, 8, _TILE), jnp.float32)),
        compiler_params=pltpu.CompilerParams(needs_layout_passes=False),
        scratch_types=[
            pltpu.VMEM((_G, 8, _TILE), jnp.float32),
            pltpu.VMEM((_G, 8, _TILE), jnp.float32),
            pltpu.SemaphoreType.DMA,
        ],
    )(_repack_body)

    gather = functools.partial(
        pl.kernel,
        mesh=mesh,
        out_type=jax.ShapeDtypeStruct((_B,), jnp.float32),
        compiler_params=pltpu.CompilerParams(
            needs_layout_passes=False, use_tc_tiling_on_sc=False),
        scratch_types=[
            pltpu.VMEM((_NCHUNK, _CHUNK), jnp.int32),
            pltpu.VMEM((_NCHUNK, _CHUNK), jnp.int32),
            pltpu.VMEM((_BPW,), jnp.int32),
            pltpu.VMEM((_BPW,), jnp.int32),
            pltpu.VMEM((_BPW,), jnp.int32),
            pltpu.VMEM((_BPW,), jnp.int32),
            pltpu.VMEM((2, _CHUNK), jnp.int32),
            pltpu.VMEM((_D * _NTAIL,), jnp.float32),
            pltpu.VMEM((_D * _NTAIL,), jnp.float32),
            pltpu.VMEM((_D * _BPW,), jnp.float32),
            pltpu.VMEM((_D * _BPW,), jnp.float32),
            pltpu.VMEM((_BPW,), jnp.float32),
            pltpu.SemaphoreType.DMA,
        ],
    )(_gather_body)

    ut = user_factors.T
    it = item_factors.T
    tail_u = ut[:, _NFULL:].reshape(-1)
    tail_i = it[:, _NFULL:].reshape(-1)
    ulin, ilin = repack(ut, it)
    return gather(user_ids.astype(jnp.int32), item_ids.astype(jnp.int32),
                  ulin.reshape(-1), ilin.reshape(-1), tail_u, tail_i)


# speedup vs baseline: 18.7627x; 1.2588x over previous
import functools

import jax
import jax.numpy as jnp
from jax import lax
from jax.experimental import pallas as pl
from jax.experimental.pallas import tpu as pltpu
from jax.experimental.pallas import tpu_sc as plsc

_B = 16384
_D = 32
_NC = 2
_NS = 16
_NW = _NC * _NS
_BPW = _B // _NW
_L = 16
_CHUNK = 128
_NCHUNK = _BPW // _CHUNK
_GROUPS = _BPW // _L

_NV = 1000000
_TILE = 128
_TPB = _NV // _TILE
_NFULL = _TPB * _TILE
_NTAIL = _NV - _NFULL
_G = 36
_CPB = _TPB // _G
_LCH = _G * _TILE
_UNITS = 4 * _CPB
_UPW = -(-_UNITS // _NW)
_NTILES = 4 * _TPB
_WORDS = _NTILES * 8 * _TILE
_KSTRIDE = _TPB * 8 * _TILE


def _repack_body(ut, it, ulin, ilin, bufu, bufi, sem):
    wid = lax.axis_index("s") * _NC + lax.axis_index("c")

    def unit(i, carry):
        u = wid + i * _NW

        @pl.when(u < _UNITS)
        def _():
            k = u // _CPB
            c = u % _CPB
            loff = pl.multiple_of(c * _LCH, _LCH)
            doff = pl.multiple_of(k * 8, 8)
            tile0 = k * _TPB + c * _G
            copies = []
            for j in range(_G):
                src = pl.ds(loff + j * _TILE, _TILE)
                copies.append(pltpu.async_copy(
                    ut.at[pl.ds(doff, 8), src], bufu.at[j], sem))
                copies.append(pltpu.async_copy(
                    it.at[pl.ds(doff, 8), src], bufi.at[j], sem))
            for cp in copies:
                cp.wait()
            wu = pltpu.async_copy(bufu, ulin.at[pl.ds(tile0, _G)], sem)
            wi = pltpu.async_copy(bufi, ilin.at[pl.ds(tile0, _G)], sem)
            wu.wait()
            wi.wait()

        return carry

    lax.fori_loop(0, _UPW, unit, 0)


def _gather_body(uid_hbm, iid_hbm, ulin, ilin, tailu_hbm, taili_hbm, out_hbm,
                 uid_v, iid_v, preu_v, prei_v, offu_v, offi_v, idx_v,
                 tailu_v, taili_v, ustage, vstage, out_v, sem):
    wid = lax.axis_index("s") * _NC + lax.axis_index("c")
    base = pl.multiple_of(wid * _BPW, _BPW)

    pltpu.sync_copy(tailu_hbm, tailu_v)
    pltpu.sync_copy(taili_hbm, taili_v)
    for c in range(_NCHUNK):
        pltpu.sync_copy(uid_hbm.at[pl.ds(base + c * _CHUNK, _CHUNK)],
                        uid_v.at[c])
        pltpu.sync_copy(iid_hbm.at[pl.ds(base + c * _CHUNK, _CHUNK)],
                        iid_v.at[c])

    def precomp(v, carry):
        s = pl.ds(pl.multiple_of(v * _L, _L), _L)
        c = v // (_CHUNK // _L)
        cs = pl.ds(pl.multiple_of((v % (_CHUNK // _L)) * _L, _L), _L)
        for ids, pre_v, off_v in ((uid_v, preu_v, offu_v),
                                  (iid_v, prei_v, offi_v)):
            r = ids[c, cs]
            off_v[s] = r - _NFULL
            rc = jnp.minimum(r, _NFULL - 1)
            pre_v[s] = ((rc >> 7) << 10) + (rc & 127)
        return carry

    lax.fori_loop(0, _BPW // _L, precomp, 0)

    def build(d, carry):
        k = d // 8
        q = d % 8
        dconst = k * _KSTRIDE + q * _TILE
        for v in range(_BPW // _L):
            s = pl.ds(pl.multiple_of(v * _L, _L), _L)
            idx_v[0, d, s] = preu_v[s] + dconst
            idx_v[1, d, s] = prei_v[s] + dconst
        return carry

    lax.fori_loop(0, _D, build, 0)

    def fetch(d, carry):
        doff = pl.multiple_of(d * _BPW, _CHUNK)
        for c in range(_NCHUNK):
            cs = pl.ds(pl.multiple_of(c * _CHUNK, _CHUNK), _CHUNK)
            dst = pl.ds(doff + c * _CHUNK, _CHUNK)
            pltpu.async_copy(ulin.at[idx_v.at[0, d, cs]], ustage.at[dst], sem)
            pltpu.async_copy(ilin.at[idx_v.at[1, d, cs]], vstage.at[dst], sem)
        return carry

    lax.fori_loop(0, _D, fetch, 0)

    pltpu.make_async_copy(out_hbm, ustage, sem).wait()
    pltpu.make_async_copy(out_hbm, vstage, sem).wait()

    iota = lax.iota(jnp.int32, _L)
    i512 = iota * _BPW
    i64 = iota * _NTAIL

    def tailfix(g, carry):
        goff = pl.multiple_of(g * _L, _L)
        offu = offu_v[pl.ds(goff, _L)]
        offi = offi_v[pl.ds(goff, _L)]
        anytail = jnp.maximum(jnp.max(offu), jnp.max(offi))

        @pl.when(anytail >= 0)
        def _():
            for j in range(_L):
                b = goff + j

                @pl.when(offu[j] >= 0)
                def _():
                    src0 = plsc.load_gather(tailu_v, [i64 + offu[j]])
                    src1 = plsc.load_gather(tailu_v, [i64 + (offu[j] + _L * _NTAIL)])
                    plsc.store_scatter(ustage, [i512 + b], src0)
                    plsc.store_scatter(ustage, [i512 + (b + _L * _BPW)], src1)

                @pl.when(offi[j] >= 0)
                def _():
                    src0 = plsc.load_gather(taili_v, [i64 + offi[j]])
                    src1 = plsc.load_gather(taili_v, [i64 + (offi[j] + _L * _NTAIL)])
                    plsc.store_scatter(vstage, [i512 + b], src0)
                    plsc.store_scatter(vstage, [i512 + (b + _L * _BPW)], src1)

        return carry

    lax.fori_loop(0, _GROUPS, tailfix, 0)

    def group(g, carry):
        goff = pl.multiple_of(g * _L, _L)
        accs = [jnp.zeros((_L,), jnp.float32) for _ in range(4)]
        for d in range(_D):
            off = pl.ds(pl.multiple_of(d * _BPW, _L) + goff, _L)
            accs[d % 4] = accs[d % 4] + ustage[off] * vstage[off]
        out_v[pl.ds(goff, _L)] = (accs[0] + accs[1]) + (accs[2] + accs[3])
        return carry

    lax.fori_loop(0, _GROUPS, group, 0)

    pltpu.sync_copy(out_v, out_hbm.at[pl.ds(base, _BPW)])


def kernel(user_ids, item_ids, user_factors, item_factors):
    mesh = plsc.VectorSubcoreMesh(core_axis_name="c", subcore_axis_name="s")

    repack = functools.partial(
        pl.kernel,
        mesh=mesh,
        out_type=(jax.ShapeDtypeStruct((_NTILES, 8, _TILE), jnp.float32),
                  jax.ShapeDtypeStruct((_NTILES, 8, _TILE), jnp.float32)),
        compiler_params=pltpu.CompilerParams(needs_layout_passes=False),
        scratch_types=[
            pltpu.VMEM((_G, 8, _TILE), jnp.float32),
            pltpu.VMEM((_G, 8, _TILE), jnp.float32),
            pltpu.SemaphoreType.DMA,
        ],
    )(_repack_body)

    gather = functools.partial(
        pl.kernel,
        mesh=mesh,
        out_type=jax.ShapeDtypeStruct((_B,), jnp.float32),
        compiler_params=pltpu.CompilerParams(
            needs_layout_passes=False, use_tc_tiling_on_sc=False),
        scratch_types=[
            pltpu.VMEM((_NCHUNK, _CHUNK), jnp.int32),
            pltpu.VMEM((_NCHUNK, _CHUNK), jnp.int32),
            pltpu.VMEM((_BPW,), jnp.int32),
            pltpu.VMEM((_BPW,), jnp.int32),
            pltpu.VMEM((_BPW,), jnp.int32),
            pltpu.VMEM((_BPW,), jnp.int32),
            pltpu.VMEM((2, _D, _BPW), jnp.int32),
            pltpu.VMEM((_D * _NTAIL,), jnp.float32),
            pltpu.VMEM((_D * _NTAIL,), jnp.float32),
            pltpu.VMEM((_D * _BPW,), jnp.float32),
            pltpu.VMEM((_D * _BPW,), jnp.float32),
            pltpu.VMEM((_BPW,), jnp.float32),
            pltpu.SemaphoreType.DMA,
        ],
    )(_gather_body)

    ut = user_factors.T
    it = item_factors.T
    tail_u = ut[:, _NFULL:].reshape(-1)
    tail_i = it[:, _NFULL:].reshape(-1)
    ulin, ilin = repack(ut, it)
    return gather(user_ids.astype(jnp.int32), item_ids.astype(jnp.int32),
                  ulin.reshape(-1), ilin.reshape(-1), tail_u, tail_i)


# speedup vs baseline: 19.8924x; 1.0602x over previous
import functools

import jax
import jax.numpy as jnp
from jax import lax
from jax.experimental import pallas as pl
from jax.experimental.pallas import tpu as pltpu
from jax.experimental.pallas import tpu_sc as plsc

_B = 16384
_D = 32
_NC = 2
_NS = 16
_NW = _NC * _NS
_BPW = _B // _NW
_L = 16
_CHUNK = 128
_NCHUNK = _BPW // _CHUNK
_GROUPS = _BPW // _L

_NV = 1000000
_TILE = 128
_TPB = _NV // _TILE
_NFULL = _TPB * _TILE
_NTAIL = _NV - _NFULL
_G = 18
_CPB = _TPB // _G
_LCH = _G * _TILE
_UNITS = 4 * _CPB
_UPW = -(-_UNITS // _NW)
_NTILES = 4 * _TPB
_WORDS = _NTILES * 8 * _TILE
_KSTRIDE = _TPB * 8 * _TILE


def _repack_body(ut, it, ulin, ilin, bufu, bufi, sem_r0, sem_r1,
                 sem_w0, sem_w1):
    wid = lax.axis_index("s") * _NC + lax.axis_index("c")
    sem_r = (sem_r0, sem_r1)
    sem_w = (sem_w0, sem_w1)

    def pair(ii, carry):
        for p in range(2):
            i = ii * 2 + p
            u = wid + i * _NW
            sr = sem_r0 if p == 0 else sem_r1
            sw = sem_w0 if p == 0 else sem_w1

            @pl.when(jnp.logical_and(i >= 2, wid + (i - 2) * _NW < _UNITS))
            def _():
                pltpu.make_async_copy(ulin.at[pl.ds(0, _G)],
                                      bufu.at[p], sw).wait()
                pltpu.make_async_copy(ilin.at[pl.ds(0, _G)],
                                      bufi.at[p], sw).wait()

            @pl.when(u < _UNITS)
            def _():
                k = u // _CPB
                c = u % _CPB
                loff = pl.multiple_of(c * _LCH, _LCH)
                doff = pl.multiple_of(k * 8, 8)
                tile0 = k * _TPB + c * _G
                for j in range(_G):
                    src = pl.ds(loff + j * _TILE, _TILE)
                    pltpu.async_copy(ut.at[pl.ds(doff, 8), src],
                                     bufu.at[p, j], sr)
                    pltpu.async_copy(it.at[pl.ds(doff, 8), src],
                                     bufi.at[p, j], sr)
                pltpu.make_async_copy(ulin.at[pl.ds(0, _G)],
                                      bufu.at[p], sr).wait()
                pltpu.make_async_copy(ilin.at[pl.ds(0, _G)],
                                      bufi.at[p], sr).wait()
                pltpu.async_copy(bufu.at[p], ulin.at[pl.ds(tile0, _G)], sw)
                pltpu.async_copy(bufi.at[p], ilin.at[pl.ds(tile0, _G)], sw)

        return carry

    lax.fori_loop(0, (_UPW + 2 + 1) // 2, pair, 0)


def _gather_body(uid_hbm, iid_hbm, ulin, ilin, tailu_hbm, taili_hbm, out_hbm,
                 uid_v, iid_v, preu_v, prei_v, offu_v, offi_v, idx_v,
                 tailu_v, taili_v, ustage, vstage, out_v, sem):
    wid = lax.axis_index("s") * _NC + lax.axis_index("c")
    base = pl.multiple_of(wid * _BPW, _BPW)

    pltpu.sync_copy(tailu_hbm, tailu_v)
    pltpu.sync_copy(taili_hbm, taili_v)
    for c in range(_NCHUNK):
        pltpu.sync_copy(uid_hbm.at[pl.ds(base + c * _CHUNK, _CHUNK)],
                        uid_v.at[c])
        pltpu.sync_copy(iid_hbm.at[pl.ds(base + c * _CHUNK, _CHUNK)],
                        iid_v.at[c])

    def precomp(v, carry):
        s = pl.ds(pl.multiple_of(v * _L, _L), _L)
        c = v // (_CHUNK // _L)
        cs = pl.ds(pl.multiple_of((v % (_CHUNK // _L)) * _L, _L), _L)
        for ids, pre_v, off_v in ((uid_v, preu_v, offu_v),
                                  (iid_v, prei_v, offi_v)):
            r = ids[c, cs]
            off_v[s] = r - _NFULL
            rc = jnp.minimum(r, _NFULL - 1)
            pre_v[s] = ((rc >> 7) << 10) + (rc & 127)
        return carry

    lax.fori_loop(0, _BPW // _L, precomp, 0)

    def build(d, carry):
        k = d // 8
        q = d % 8
        dconst = k * _KSTRIDE + q * _TILE
        for v in range(_BPW // _L):
            s = pl.ds(pl.multiple_of(v * _L, _L), _L)
            idx_v[0, d, s] = preu_v[s] + dconst
            idx_v[1, d, s] = prei_v[s] + dconst
        return carry

    lax.fori_loop(0, _D, build, 0)

    def fetch(d, carry):
        doff = pl.multiple_of(d * _BPW, _CHUNK)
        for c in range(_NCHUNK):
            cs = pl.ds(pl.multiple_of(c * _CHUNK, _CHUNK), _CHUNK)
            dst = pl.ds(doff + c * _CHUNK, _CHUNK)
            pltpu.async_copy(ulin.at[idx_v.at[0, d, cs]], ustage.at[dst], sem)
            pltpu.async_copy(ilin.at[idx_v.at[1, d, cs]], vstage.at[dst], sem)
        return carry

    lax.fori_loop(0, _D, fetch, 0)

    pltpu.make_async_copy(out_hbm, ustage, sem).wait()
    pltpu.make_async_copy(out_hbm, vstage, sem).wait()

    iota = lax.iota(jnp.int32, _L)
    i512 = iota * _BPW
    i64 = iota * _NTAIL

    def tailfix(g, carry):
        goff = pl.multiple_of(g * _L, _L)
        offu = offu_v[pl.ds(goff, _L)]
        offi = offi_v[pl.ds(goff, _L)]
        anytail = jnp.maximum(jnp.max(offu), jnp.max(offi))

        @pl.when(anytail >= 0)
        def _():
            for j in range(_L):
                b = goff + j

                @pl.when(offu[j] >= 0)
                def _():
                    src0 = plsc.load_gather(tailu_v, [i64 + offu[j]])
                    src1 = plsc.load_gather(tailu_v, [i64 + (offu[j] + _L * _NTAIL)])
                    plsc.store_scatter(ustage, [i512 + b], src0)
                    plsc.store_scatter(ustage, [i512 + (b + _L * _BPW)], src1)

                @pl.when(offi[j] >= 0)
                def _():
                    src0 = plsc.load_gather(taili_v, [i64 + offi[j]])
                    src1 = plsc.load_gather(taili_v, [i64 + (offi[j] + _L * _NTAIL)])
                    plsc.store_scatter(vstage, [i512 + b], src0)
                    plsc.store_scatter(vstage, [i512 + (b + _L * _BPW)], src1)

        return carry

    lax.fori_loop(0, _GROUPS, tailfix, 0)

    def group(g, carry):
        goff = pl.multiple_of(g * _L, _L)
        accs = [jnp.zeros((_L,), jnp.float32) for _ in range(4)]
        for d in range(_D):
            off = pl.ds(pl.multiple_of(d * _BPW, _L) + goff, _L)
            accs[d % 4] = accs[d % 4] + ustage[off] * vstage[off]
        out_v[pl.ds(goff, _L)] = (accs[0] + accs[1]) + (accs[2] + accs[3])
        return carry

    lax.fori_loop(0, _GROUPS, group, 0)

    pltpu.sync_copy(out_v, out_hbm.at[pl.ds(base, _BPW)])


def kernel(user_ids, item_ids, user_factors, item_factors):
    mesh = plsc.VectorSubcoreMesh(core_axis_name="c", subcore_axis_name="s")

    repack = functools.partial(
        pl.kernel,
        mesh=mesh,
        out_type=(jax.ShapeDtypeStruct((_NTILES, 8, _TILE), jnp.float32),
                  jax.ShapeDtypeStruct((_NTILES, 8, _TILE), jnp.float32)),
        compiler_params=pltpu.CompilerParams(needs_layout_passes=False),
        scratch_types=[
            pltpu.VMEM((2, _G, 8, _TILE), jnp.float32),
            pltpu.VMEM((2, _G, 8, _TILE), jnp.float32),
            pltpu.SemaphoreType.DMA,
            pltpu.SemaphoreType.DMA,
            pltpu.SemaphoreType.DMA,
            pltpu.SemaphoreType.DMA,
        ],
    )(_repack_body)

    gather = functools.partial(
        pl.kernel,
        mesh=mesh,
        out_type=jax.ShapeDtypeStruct((_B,), jnp.float32),
        compiler_params=pltpu.CompilerParams(
            needs_layout_passes=False, use_tc_tiling_on_sc=False),
        scratch_types=[
            pltpu.VMEM((_NCHUNK, _CHUNK), jnp.int32),
            pltpu.VMEM((_NCHUNK, _CHUNK), jnp.int32),
            pltpu.VMEM((_BPW,), jnp.int32),
            pltpu.VMEM((_BPW,), jnp.int32),
            pltpu.VMEM((_BPW,), jnp.int32),
            pltpu.VMEM((_BPW,), jnp.int32),
            pltpu.VMEM((2, _D, _BPW), jnp.int32),
            pltpu.VMEM((_D * _NTAIL,), jnp.float32),
            pltpu.VMEM((_D * _NTAIL,), jnp.float32),
            pltpu.VMEM((_D * _BPW,), jnp.float32),
            pltpu.VMEM((_D * _BPW,), jnp.float32),
            pltpu.VMEM((_BPW,), jnp.float32),
            pltpu.SemaphoreType.DMA,
        ],
    )(_gather_body)

    ut = user_factors.T
    it = item_factors.T
    tail_u = ut[:, _NFULL:].reshape(-1)
    tail_i = it[:, _NFULL:].reshape(-1)
    ulin, ilin = repack(ut, it)
    return gather(user_ids.astype(jnp.int32), item_ids.astype(jnp.int32),
                  ulin.reshape(-1), ilin.reshape(-1), tail_u, tail_i)


# speedup vs baseline: 20.0748x; 1.0092x over previous
import functools

import jax
import jax.numpy as jnp
from jax import lax
from jax.experimental import pallas as pl
from jax.experimental.pallas import tpu as pltpu
from jax.experimental.pallas import tpu_sc as plsc

_B = 16384
_D = 32
_NC = 2
_NS = 16
_NW = _NC * _NS
_BPW = _B // _NW
_L = 16
_CHUNK = 128
_NCHUNK = _BPW // _CHUNK
_GROUPS = _BPW // _L

_NV = 1000000
_TILE = 128
_TPB = _NV // _TILE
_NFULL = _TPB * _TILE
_NTAIL = _NV - _NFULL
_G = 28
_CPB = _TPB // _G
_LCH = _G * _TILE
_UNITS = 4 * _CPB
_UPW = -(-_UNITS // _NW)
_NTILES = 4 * _TPB
_WORDS = _NTILES * 8 * _TILE
_KSTRIDE = _TPB * 8 * _TILE


def _repack_body(ut, it, ulin, ilin, bufu, bufi, sem_r0, sem_r1,
                 sem_w0, sem_w1):
    wid = lax.axis_index("s") * _NC + lax.axis_index("c")
    sem_r = (sem_r0, sem_r1)
    sem_w = (sem_w0, sem_w1)

    def pair(ii, carry):
        for p in range(2):
            i = ii * 2 + p
            u = wid + i * _NW
            sr = sem_r0 if p == 0 else sem_r1
            sw = sem_w0 if p == 0 else sem_w1

            @pl.when(jnp.logical_and(i >= 2, wid + (i - 2) * _NW < _UNITS))
            def _():
                pltpu.make_async_copy(ulin.at[pl.ds(0, _G)],
                                      bufu.at[p], sw).wait()
                pltpu.make_async_copy(ilin.at[pl.ds(0, _G)],
                                      bufi.at[p], sw).wait()

            @pl.when(u < _UNITS)
            def _():
                k = u // _CPB
                c = u % _CPB
                loff = pl.multiple_of(c * _LCH, _LCH)
                doff = pl.multiple_of(k * 8, 8)
                tile0 = k * _TPB + c * _G
                for j in range(_G):
                    src = pl.ds(loff + j * _TILE, _TILE)
                    pltpu.async_copy(ut.at[pl.ds(doff, 8), src],
                                     bufu.at[p, j], sr)
                    pltpu.async_copy(it.at[pl.ds(doff, 8), src],
                                     bufi.at[p, j], sr)
                pltpu.make_async_copy(ulin.at[pl.ds(0, _G)],
                                      bufu.at[p], sr).wait()
                pltpu.make_async_copy(ilin.at[pl.ds(0, _G)],
                                      bufi.at[p], sr).wait()
                pltpu.async_copy(bufu.at[p], ulin.at[pl.ds(tile0, _G)], sw)
                pltpu.async_copy(bufi.at[p], ilin.at[pl.ds(tile0, _G)], sw)

        return carry

    lax.fori_loop(0, (_UPW + 2 + 1) // 2, pair, 0)


def _gather_body(uid_hbm, iid_hbm, ulin, ilin, tailu_hbm, taili_hbm, out_hbm,
                 uid_v, iid_v, preu_v, prei_v, offu_v, offi_v, idx_v,
                 tailu_v, taili_v, ustage, vstage, out_v, sem):
    wid = lax.axis_index("s") * _NC + lax.axis_index("c")
    base = pl.multiple_of(wid * _BPW, _BPW)

    pltpu.sync_copy(tailu_hbm, tailu_v)
    pltpu.sync_copy(taili_hbm, taili_v)
    for c in range(_NCHUNK):
        pltpu.sync_copy(uid_hbm.at[pl.ds(base + c * _CHUNK, _CHUNK)],
                        uid_v.at[c])
        pltpu.sync_copy(iid_hbm.at[pl.ds(base + c * _CHUNK, _CHUNK)],
                        iid_v.at[c])

    def precomp(v, carry):
        s = pl.ds(pl.multiple_of(v * _L, _L), _L)
        c = v // (_CHUNK // _L)
        cs = pl.ds(pl.multiple_of((v % (_CHUNK // _L)) * _L, _L), _L)
        for ids, pre_v, off_v in ((uid_v, preu_v, offu_v),
                                  (iid_v, prei_v, offi_v)):
            r = ids[c, cs]
            off_v[s] = r - _NFULL
            rc = jnp.minimum(r, _NFULL - 1)
            pre_v[s] = ((rc >> 7) << 10) + (rc & 127)
        return carry

    lax.fori_loop(0, _BPW // _L, precomp, 0)

    def build(d, carry):
        k = d // 8
        q = d % 8
        dconst = k * _KSTRIDE + q * _TILE
        for v in range(_BPW // _L):
            s = pl.ds(pl.multiple_of(v * _L, _L), _L)
            idx_v[0, d, s] = preu_v[s] + dconst
            idx_v[1, d, s] = prei_v[s] + dconst
        return carry

    lax.fori_loop(0, _D, build, 0)

    def fetch(d, carry):
        dst = pl.ds(pl.multiple_of(d * _BPW, _BPW), _BPW)
        pltpu.async_copy(ulin.at[idx_v.at[0, d]], ustage.at[dst], sem)
        pltpu.async_copy(ilin.at[idx_v.at[1, d]], vstage.at[dst], sem)
        return carry

    lax.fori_loop(0, _D, fetch, 0)

    pltpu.make_async_copy(out_hbm, ustage, sem).wait()
    pltpu.make_async_copy(out_hbm, vstage, sem).wait()

    iota = lax.iota(jnp.int32, _L)
    i512 = iota * _BPW
    i64 = iota * _NTAIL

    def tailfix(g, carry):
        goff = pl.multiple_of(g * _L, _L)
        offu = offu_v[pl.ds(goff, _L)]
        offi = offi_v[pl.ds(goff, _L)]
        anytail = jnp.maximum(jnp.max(offu), jnp.max(offi))

        @pl.when(anytail >= 0)
        def _():
            for j in range(_L):
                b = goff + j

                @pl.when(offu[j] >= 0)
                def _():
                    src0 = plsc.load_gather(tailu_v, [i64 + offu[j]])
                    src1 = plsc.load_gather(tailu_v, [i64 + (offu[j] + _L * _NTAIL)])
                    plsc.store_scatter(ustage, [i512 + b], src0)
                    plsc.store_scatter(ustage, [i512 + (b + _L * _BPW)], src1)

                @pl.when(offi[j] >= 0)
                def _():
                    src0 = plsc.load_gather(taili_v, [i64 + offi[j]])
                    src1 = plsc.load_gather(taili_v, [i64 + (offi[j] + _L * _NTAIL)])
                    plsc.store_scatter(vstage, [i512 + b], src0)
                    plsc.store_scatter(vstage, [i512 + (b + _L * _BPW)], src1)

        return carry

    lax.fori_loop(0, _GROUPS, tailfix, 0)

    def group(g, carry):
        goff = pl.multiple_of(g * _L, _L)
        accs = [jnp.zeros((_L,), jnp.float32) for _ in range(4)]
        for d in range(_D):
            off = pl.ds(pl.multiple_of(d * _BPW, _L) + goff, _L)
            accs[d % 4] = accs[d % 4] + ustage[off] * vstage[off]
        out_v[pl.ds(goff, _L)] = (accs[0] + accs[1]) + (accs[2] + accs[3])
        return carry

    lax.fori_loop(0, _GROUPS, group, 0)

    pltpu.sync_copy(out_v, out_hbm.at[pl.ds(base, _BPW)])


def kernel(user_ids, item_ids, user_factors, item_factors):
    mesh = plsc.VectorSubcoreMesh(core_axis_name="c", subcore_axis_name="s")

    repack = functools.partial(
        pl.kernel,
        mesh=mesh,
        out_type=(jax.ShapeDtypeStruct((_NTILES, 8, _TILE), jnp.float32),
                  jax.ShapeDtypeStruct((_NTILES, 8, _TILE), jnp.float32)),
        compiler_params=pltpu.CompilerParams(needs_layout_passes=False),
        scratch_types=[
            pltpu.VMEM((2, _G, 8, _TILE), jnp.float32),
            pltpu.VMEM((2, _G, 8, _TILE), jnp.float32),
            pltpu.SemaphoreType.DMA,
            pltpu.SemaphoreType.DMA,
            pltpu.SemaphoreType.DMA,
            pltpu.SemaphoreType.DMA,
        ],
    )(_repack_body)

    gather = functools.partial(
        pl.kernel,
        mesh=mesh,
        out_type=jax.ShapeDtypeStruct((_B,), jnp.float32),
        compiler_params=pltpu.CompilerParams(
            needs_layout_passes=False, use_tc_tiling_on_sc=False),
        scratch_types=[
            pltpu.VMEM((_NCHUNK, _CHUNK), jnp.int32),
            pltpu.VMEM((_NCHUNK, _CHUNK), jnp.int32),
            pltpu.VMEM((_BPW,), jnp.int32),
            pltpu.VMEM((_BPW,), jnp.int32),
            pltpu.VMEM((_BPW,), jnp.int32),
            pltpu.VMEM((_BPW,), jnp.int32),
            pltpu.VMEM((2, _D, _BPW), jnp.int32),
            pltpu.VMEM((_D * _NTAIL,), jnp.float32),
            pltpu.VMEM((_D * _NTAIL,), jnp.float32),
            pltpu.VMEM((_D * _BPW,), jnp.float32),
            pltpu.VMEM((_D * _BPW,), jnp.float32),
            pltpu.VMEM((_BPW,), jnp.float32),
            pltpu.SemaphoreType.DMA,
        ],
    )(_gather_body)

    ut = user_factors.T
    it = item_factors.T
    tail_u = ut[:, _NFULL:].reshape(-1)
    tail_i = it[:, _NFULL:].reshape(-1)
    ulin, ilin = repack(ut, it)
    return gather(user_ids.astype(jnp.int32), item_ids.astype(jnp.int32),
                  ulin.reshape(-1), ilin.reshape(-1), tail_u, tail_i)


# speedup vs baseline: 20.7918x; 1.0357x over previous
import functools

import jax
import jax.numpy as jnp
from jax import lax
from jax.experimental import pallas as pl
from jax.experimental.pallas import tpu as pltpu
from jax.experimental.pallas import tpu_sc as plsc

_B = 16384
_D = 32
_NC = 2
_NS = 16
_NW = _NC * _NS
_BPW = _B // _NW
_L = 16
_CHUNK = 128
_NCHUNK = _BPW // _CHUNK
_GROUPS = _BPW // _L

_NV = 1000000
_TILE = 128
_TPB = _NV // _TILE
_NFULL = _TPB * _TILE
_NTAIL = _NV - _NFULL
_G = 28
_CPB = _TPB // _G
_LCH = _G * _TILE
_UNITS = 4 * _CPB
_UPW = -(-_UNITS // _NW)
_NTILES = 4 * _TPB
_WORDS = _NTILES * 8 * _TILE
_KSTRIDE = _TPB * 8 * _TILE


def _repack_body(ut, it, ulin, ilin, bufu, bufi, sem_r0, sem_r1,
                 sem_w0, sem_w1):
    wid = lax.axis_index("s") * _NC + lax.axis_index("c")
    sem_r = (sem_r0, sem_r1)
    sem_w = (sem_w0, sem_w1)

    def pair(ii, carry):
        for p in range(2):
            i = ii * 2 + p
            u = wid + i * _NW
            sr = sem_r0 if p == 0 else sem_r1
            sw = sem_w0 if p == 0 else sem_w1

            @pl.when(jnp.logical_and(i >= 2, wid + (i - 2) * _NW < _UNITS))
            def _():
                pltpu.make_async_copy(ulin.at[pl.ds(0, _G)],
                                      bufu.at[p], sw).wait()
                pltpu.make_async_copy(ilin.at[pl.ds(0, _G)],
                                      bufi.at[p], sw).wait()

            @pl.when(u < _UNITS)
            def _():
                k = u // _CPB
                c = u % _CPB
                loff = pl.multiple_of(c * _LCH, _LCH)
                doff = pl.multiple_of(k * 8, 8)
                tile0 = k * _TPB + c * _G
                for j in range(_G):
                    src = pl.ds(loff + j * _TILE, _TILE)
                    pltpu.async_copy(ut.at[pl.ds(doff, 8), src],
                                     bufu.at[p, j], sr)
                    pltpu.async_copy(it.at[pl.ds(doff, 8), src],
                                     bufi.at[p, j], sr)
                pltpu.make_async_copy(ulin.at[pl.ds(0, _G)],
                                      bufu.at[p], sr).wait()
                pltpu.make_async_copy(ilin.at[pl.ds(0, _G)],
                                      bufi.at[p], sr).wait()
                pltpu.async_copy(bufu.at[p], ulin.at[pl.ds(tile0, _G)], sw)
                pltpu.async_copy(bufi.at[p], ilin.at[pl.ds(tile0, _G)], sw)

        return carry

    lax.fori_loop(0, (_UPW + 2 + 1) // 2, pair, 0)


def _gather_body(uid_hbm, iid_hbm, ulin, ilin, tailu_hbm, taili_hbm, out_hbm,
                 uid_v, iid_v, preu_v, prei_v, offu_v, offi_v, idx_v,
                 tailu_v, taili_v, ustage, vstage, out_v, sem):
    wid = lax.axis_index("s") * _NC + lax.axis_index("c")
    base = pl.multiple_of(wid * _BPW, _BPW)

    pltpu.sync_copy(tailu_hbm, tailu_v)
    pltpu.sync_copy(taili_hbm, taili_v)
    for c in range(_NCHUNK):
        pltpu.sync_copy(uid_hbm.at[pl.ds(base + c * _CHUNK, _CHUNK)],
                        uid_v.at[c])
        pltpu.sync_copy(iid_hbm.at[pl.ds(base + c * _CHUNK, _CHUNK)],
                        iid_v.at[c])

    def precomp(v, carry):
        s = pl.ds(pl.multiple_of(v * _L, _L), _L)
        c = v // (_CHUNK // _L)
        cs = pl.ds(pl.multiple_of((v % (_CHUNK // _L)) * _L, _L), _L)
        for ids, pre_v, off_v in ((uid_v, preu_v, offu_v),
                                  (iid_v, prei_v, offi_v)):
            r = ids[c, cs]
            off_v[s] = r - _NFULL
            rc = jnp.minimum(r, _NFULL - 1)
            pre_v[s] = ((rc >> 7) << 10) + (rc & 127)
        return carry

    lax.fori_loop(0, _BPW // _L, precomp, 0)

    def fetch(d, carry):
        k = d // 8
        q = d % 8
        dconst = k * _KSTRIDE + q * _TILE
        for v in range(_BPW // _L):
            s = pl.ds(pl.multiple_of(v * _L, _L), _L)
            idx_v[0, d, s] = preu_v[s] + dconst
            idx_v[1, d, s] = prei_v[s] + dconst
        dst = pl.ds(pl.multiple_of(d * _BPW, _BPW), _BPW)
        pltpu.async_copy(ulin.at[idx_v.at[0, d]], ustage.at[dst], sem)
        pltpu.async_copy(ilin.at[idx_v.at[1, d]], vstage.at[dst], sem)
        return carry

    lax.fori_loop(0, _D, fetch, 0)

    pltpu.make_async_copy(out_hbm, ustage, sem).wait()
    pltpu.make_async_copy(out_hbm, vstage, sem).wait()

    iota = lax.iota(jnp.int32, _L)
    i512 = iota * _BPW
    i64 = iota * _NTAIL

    def tailfix(g, carry):
        goff = pl.multiple_of(g * _L, _L)
        offu = offu_v[pl.ds(goff, _L)]
        offi = offi_v[pl.ds(goff, _L)]
        anytail = jnp.maximum(jnp.max(offu), jnp.max(offi))

        @pl.when(anytail >= 0)
        def _():
            for j in range(_L):
                b = goff + j

                @pl.when(offu[j] >= 0)
                def _():
                    src0 = plsc.load_gather(tailu_v, [i64 + offu[j]])
                    src1 = plsc.load_gather(tailu_v, [i64 + (offu[j] + _L * _NTAIL)])
                    plsc.store_scatter(ustage, [i512 + b], src0)
                    plsc.store_scatter(ustage, [i512 + (b + _L * _BPW)], src1)

                @pl.when(offi[j] >= 0)
                def _():
                    src0 = plsc.load_gather(taili_v, [i64 + offi[j]])
                    src1 = plsc.load_gather(taili_v, [i64 + (offi[j] + _L * _NTAIL)])
                    plsc.store_scatter(vstage, [i512 + b], src0)
                    plsc.store_scatter(vstage, [i512 + (b + _L * _BPW)], src1)

        return carry

    lax.fori_loop(0, _GROUPS, tailfix, 0)

    def group(g, carry):
        goff = pl.multiple_of(g * _L, _L)
        accs = [jnp.zeros((_L,), jnp.float32) for _ in range(4)]
        for d in range(_D):
            off = pl.ds(pl.multiple_of(d * _BPW, _L) + goff, _L)
            accs[d % 4] = accs[d % 4] + ustage[off] * vstage[off]
        out_v[pl.ds(goff, _L)] = (accs[0] + accs[1]) + (accs[2] + accs[3])
        return carry

    lax.fori_loop(0, _GROUPS, group, 0)

    pltpu.sync_copy(out_v, out_hbm.at[pl.ds(base, _BPW)])


def kernel(user_ids, item_ids, user_factors, item_factors):
    mesh = plsc.VectorSubcoreMesh(core_axis_name="c", subcore_axis_name="s")

    repack = functools.partial(
        pl.kernel,
        mesh=mesh,
        out_type=(jax.ShapeDtypeStruct((_NTILES, 8, _TILE), jnp.float32),
                  jax.ShapeDtypeStruct((_NTILES, 8, _TILE), jnp.float32)),
        compiler_params=pltpu.CompilerParams(needs_layout_passes=False),
        scratch_types=[
            pltpu.VMEM((2, _G, 8, _TILE), jnp.float32),
            pltpu.VMEM((2, _G, 8, _TILE), jnp.float32),
            pltpu.SemaphoreType.DMA,
            pltpu.SemaphoreType.DMA,
            pltpu.SemaphoreType.DMA,
            pltpu.SemaphoreType.DMA,
        ],
    )(_repack_body)

    gather = functools.partial(
        pl.kernel,
        mesh=mesh,
        out_type=jax.ShapeDtypeStruct((_B,), jnp.float32),
        compiler_params=pltpu.CompilerParams(
            needs_layout_passes=False, use_tc_tiling_on_sc=False),
        scratch_types=[
            pltpu.VMEM((_NCHUNK, _CHUNK), jnp.int32),
            pltpu.VMEM((_NCHUNK, _CHUNK), jnp.int32),
            pltpu.VMEM((_BPW,), jnp.int32),
            pltpu.VMEM((_BPW,), jnp.int32),
            pltpu.VMEM((_BPW,), jnp.int32),
            pltpu.VMEM((_BPW,), jnp.int32),
            pltpu.VMEM((2, _D, _BPW), jnp.int32),
            pltpu.VMEM((_D * _NTAIL,), jnp.float32),
            pltpu.VMEM((_D * _NTAIL,), jnp.float32),
            pltpu.VMEM((_D * _BPW,), jnp.float32),
            pltpu.VMEM((_D * _BPW,), jnp.float32),
            pltpu.VMEM((_BPW,), jnp.float32),
            pltpu.SemaphoreType.DMA,
        ],
    )(_gather_body)

    ut = user_factors.T
    it = item_factors.T
    tail_u = ut[:, _NFULL:].reshape(-1)
    tail_i = it[:, _NFULL:].reshape(-1)
    ulin, ilin = repack(ut, it)
    return gather(user_ids.astype(jnp.int32), item_ids.astype(jnp.int32),
                  ulin.reshape(-1), ilin.reshape(-1), tail_u, tail_i)
